# Initial kernel scaffold; baseline (speedup 1.0000x reference)
import jax
import jax.numpy as jnp
from jax.experimental import pallas as pl


def kernel(user_id, item_id, score, user_w, a_w, b_w):
    raise NotImplementedError("write your pallas kernel here")


# speedup vs baseline: 1.1486x; 1.1486x over previous
import functools

import jax
import jax.numpy as jnp
from jax import lax
from jax.experimental import pallas as pl
from jax.experimental.pallas import tpu as pltpu
from jax.experimental.pallas import tpu_sc as plsc

B = 16384
_info = plsc.get_sparse_core_info()
NC, NS, L = _info.num_cores, _info.num_subcores, _info.num_lanes
NW = NC * NS
BPW = B // NW
IDX_CHUNK = 128

_mesh = plsc.VectorSubcoreMesh(core_axis_name="c", subcore_axis_name="s")


@functools.partial(
    pl.kernel,
    mesh=_mesh,
    out_type=jax.ShapeDtypeStruct((B,), jnp.float32),
    scratch_types=[
        pltpu.VMEM((BPW,), jnp.int32),
        pltpu.VMEM((BPW,), jnp.int32),
        pltpu.VMEM((BPW,), jnp.float32),
        pltpu.VMEM((BPW,), jnp.float32),
        pltpu.VMEM((BPW,), jnp.float32),
        pltpu.VMEM((BPW,), jnp.float32),
        pltpu.SemaphoreType.DMA,
    ],
)
def _sc_pred(uid_hbm, iid_hbm, uw_hbm, aw_hbm, bw_hbm, out_hbm,
             uid_v, iid_v, u_v, a_v, b_v, p_v, sem):
    wid = lax.axis_index("s") * NC + lax.axis_index("c")
    base = wid * BPW
    pltpu.sync_copy(uid_hbm.at[pl.ds(base, BPW)], uid_v)
    pltpu.sync_copy(iid_hbm.at[pl.ds(base, BPW)], iid_v)
    copies = []
    for j in range(BPW // IDX_CHUNK):
        sl = pl.ds(j * IDX_CHUNK, IDX_CHUNK)
        copies.append(pltpu.async_copy(uw_hbm.at[uid_v.at[sl]], u_v.at[sl], sem))
        copies.append(pltpu.async_copy(aw_hbm.at[iid_v.at[sl]], a_v.at[sl], sem))
        copies.append(pltpu.async_copy(bw_hbm.at[iid_v.at[sl]], b_v.at[sl], sem))
    for c in copies:
        c.wait()
    for k in range(BPW // L):
        s16 = pl.ds(k * L, L)
        z = a_v[s16] * (u_v[s16] - b_v[s16])
        p_v[s16] = 1.0 / (1.0 + jnp.exp(-z))
    pltpu.sync_copy(p_v, out_hbm.at[pl.ds(base, BPW)])


def _loss_body(p_ref, s_ref, o_ref):
    p = p_ref[...]
    s = s_ref[...]
    log_p = jnp.maximum(jnp.log(p), -100.0)
    log_1mp = jnp.maximum(jnp.log(1.0 - p), -100.0)
    o_ref[0, 0] = -jnp.sum(s * log_p + (1.0 - s) * log_1mp) * (1.0 / B)


_tc_loss = pl.pallas_call(
    _loss_body,
    out_shape=jax.ShapeDtypeStruct((1, 1), jnp.float32),
    out_specs=pl.BlockSpec(memory_space=pltpu.SMEM),
)


def kernel(user_id, item_id, score, user_w, a_w, b_w):
    pred = _sc_pred(user_id.astype(jnp.int32), item_id,
                    user_w[:, 0], a_w[:, 0], b_w[:, 0])
    loss = _tc_loss(pred.reshape(128, 128), score.reshape(128, 128))[0, 0]
    return pred, loss


# speedup vs baseline: 3.1193x; 2.7158x over previous
import functools

import jax
import jax.numpy as jnp
from jax import lax
from jax.experimental import pallas as pl
from jax.experimental.pallas import tpu as pltpu
from jax.experimental.pallas import tpu_sc as plsc

B = 16384
_info = plsc.get_sparse_core_info()
NC, NS, L = _info.num_cores, _info.num_subcores, _info.num_lanes
NW = NC * NS
BPW = B // NW
IDX_CHUNK = 128

_mesh = plsc.VectorSubcoreMesh(core_axis_name="c", subcore_axis_name="s")


@functools.partial(
    pl.kernel,
    mesh=_mesh,
    out_type=jax.ShapeDtypeStruct((B,), jnp.float32),
    scratch_types=[
        pltpu.VMEM((BPW,), jnp.int32),
        pltpu.VMEM((BPW,), jnp.int32),
        pltpu.VMEM((BPW,), jnp.float32),
        pltpu.VMEM((BPW,), jnp.float32),
        pltpu.VMEM((BPW,), jnp.float32),
        pltpu.VMEM((BPW,), jnp.float32),
        pltpu.SemaphoreType.DMA,
    ],
)
def _sc_pred(uid_hbm, iid_hbm, uw_hbm, aw_hbm, bw_hbm, out_hbm,
             uid_v, iid_v, u_v, a_v, b_v, p_v, sem):
    wid = lax.axis_index("s") * NC + lax.axis_index("c")
    base = wid * BPW
    uw1 = uw_hbm.at[0]
    aw1 = aw_hbm.at[0]
    bw1 = bw_hbm.at[0]
    pltpu.sync_copy(uid_hbm.at[pl.ds(base, BPW)], uid_v)
    pltpu.sync_copy(iid_hbm.at[pl.ds(base, BPW)], iid_v)
    copies = []
    for j in range(BPW // IDX_CHUNK):
        sl = pl.ds(j * IDX_CHUNK, IDX_CHUNK)
        copies.append(pltpu.async_copy(uw1.at[uid_v.at[sl]], u_v.at[sl], sem))
        copies.append(pltpu.async_copy(aw1.at[iid_v.at[sl]], a_v.at[sl], sem))
        copies.append(pltpu.async_copy(bw1.at[iid_v.at[sl]], b_v.at[sl], sem))
    for c in copies:
        c.wait()
    for k in range(BPW // L):
        s16 = pl.ds(k * L, L)
        z = a_v[s16] * (u_v[s16] - b_v[s16])
        p_v[s16] = 1.0 / (1.0 + jnp.exp(-z))
    pltpu.sync_copy(p_v, out_hbm.at[pl.ds(base, BPW)])


def _loss_body(p_ref, s_ref, o_ref):
    p = p_ref[...]
    s = s_ref[...]
    log_p = jnp.maximum(jnp.log(p), -100.0)
    log_1mp = jnp.maximum(jnp.log(1.0 - p), -100.0)
    o_ref[0, 0] = -jnp.sum(s * log_p + (1.0 - s) * log_1mp) * (1.0 / B)


_tc_loss = pl.pallas_call(
    _loss_body,
    out_shape=jax.ShapeDtypeStruct((1, 1), jnp.float32),
    out_specs=pl.BlockSpec(memory_space=pltpu.SMEM),
)


def kernel(user_id, item_id, score, user_w, a_w, b_w):
    pred = _sc_pred(user_id.astype(jnp.int32), item_id,
                    user_w.T, a_w.T, b_w.T)
    loss = _tc_loss(pred.reshape(128, 128), score.reshape(128, 128))[0, 0]
    return pred, loss


# speedup vs baseline: 3.2394x; 1.0385x over previous
import functools

import jax
import jax.numpy as jnp
from jax import lax
from jax.experimental import pallas as pl
from jax.experimental.pallas import tpu as pltpu
from jax.experimental.pallas import tpu_sc as plsc

B = 16384
_info = plsc.get_sparse_core_info()
NC, NS, L = _info.num_cores, _info.num_subcores, _info.num_lanes
NW = NC * NS
BPW = B // NW
IDX_CHUNK = 128

_mesh = plsc.VectorSubcoreMesh(core_axis_name="c", subcore_axis_name="s")


@functools.partial(
    pl.kernel,
    mesh=_mesh,
    out_type=jax.ShapeDtypeStruct((B,), jnp.float32),
    scratch_types=[
        pltpu.VMEM((BPW,), jnp.int32),
        pltpu.VMEM((BPW,), jnp.int32),
        pltpu.VMEM((BPW,), jnp.float32),
        pltpu.VMEM((BPW,), jnp.float32),
        pltpu.VMEM((BPW,), jnp.float32),
        pltpu.VMEM((BPW,), jnp.float32),
        pltpu.SemaphoreType.DMA,
        pltpu.SemaphoreType.DMA,
        pltpu.SemaphoreType.DMA,
        pltpu.SemaphoreType.DMA,
        pltpu.SemaphoreType.DMA,
        pltpu.SemaphoreType.DMA,
    ],
)
def _sc_pred(uid_hbm, iid_hbm, uw_hbm, aw_hbm, bw_hbm, out_hbm,
             uid_v, iid_v, u_v, a_v, b_v, p_v, sem_u, sem_i, *gsems):
    wid = lax.axis_index("s") * NC + lax.axis_index("c")
    base = wid * BPW
    uw1 = uw_hbm.at[0]
    aw1 = aw_hbm.at[0]
    bw1 = bw_hbm.at[0]
    cu = pltpu.async_copy(uid_hbm.at[pl.ds(base, BPW)], uid_v, sem_u)
    ci = pltpu.async_copy(iid_hbm.at[pl.ds(base, BPW)], iid_v, sem_i)
    nchunks = BPW // IDX_CHUNK
    copies = []
    cu.wait()
    for j in range(nchunks):
        sl = pl.ds(j * IDX_CHUNK, IDX_CHUNK)
        copies.append(pltpu.async_copy(uw1.at[uid_v.at[sl]], u_v.at[sl], gsems[j]))
    ci.wait()
    for j in range(nchunks):
        sl = pl.ds(j * IDX_CHUNK, IDX_CHUNK)
        copies.append(pltpu.async_copy(aw1.at[iid_v.at[sl]], a_v.at[sl], gsems[j]))
        copies.append(pltpu.async_copy(bw1.at[iid_v.at[sl]], b_v.at[sl], gsems[j]))
    for j in range(nchunks):
        copies[j].wait()
        copies[nchunks + 2 * j].wait()
        copies[nchunks + 2 * j + 1].wait()
        for k in range(IDX_CHUNK // L):
            s16 = pl.ds(j * IDX_CHUNK + k * L, L)
            z = a_v[s16] * (u_v[s16] - b_v[s16])
            p_v[s16] = 1.0 / (1.0 + jnp.exp(-z))
    pltpu.sync_copy(p_v, out_hbm.at[pl.ds(base, BPW)])


def _loss_body(p_ref, s_ref, o_ref):
    p = p_ref[...]
    s = s_ref[...]
    log_p = jnp.maximum(jnp.log(p), -100.0)
    log_1mp = jnp.maximum(jnp.log(1.0 - p), -100.0)
    o_ref[0, 0] = -jnp.sum(s * log_p + (1.0 - s) * log_1mp) * (1.0 / B)


_tc_loss = pl.pallas_call(
    _loss_body,
    out_shape=jax.ShapeDtypeStruct((1, 1), jnp.float32),
    out_specs=pl.BlockSpec(memory_space=pltpu.SMEM),
)


def kernel(user_id, item_id, score, user_w, a_w, b_w):
    pred = _sc_pred(user_id.astype(jnp.int32), item_id,
                    user_w.T, a_w.T, b_w.T)
    loss = _tc_loss(pred.reshape(128, 128), score.reshape(128, 128))[0, 0]
    return pred, loss
